# baseline (speedup 1.0000x reference)
import functools

import jax
import jax.numpy as jnp
from jax import lax
from jax.experimental import pallas as pl
from jax.experimental.pallas import tpu as pltpu
from jax.experimental.pallas import tpu_sc as plsc

NC = 2
NS = 16
NW = NC * NS


def _sc_mesh():
  return plsc.VectorSubcoreMesh(
      core_axis_name="c", subcore_axis_name="s", num_cores=NC,
      num_subcores=NS)


_SC_PARAMS = pltpu.CompilerParams(use_tc_tiling_on_sc=False)


@functools.lru_cache(maxsize=None)
def _deg_call(N, E):
  EW = E // NW
  CH = 80
  NCHUNK = EW // CH
  RPT = N // NS
  RCH = 125
  NR = RPT // RCH

  @functools.partial(
      pl.kernel, mesh=_sc_mesh(), compiler_params=_SC_PARAMS,
      out_type=jax.ShapeDtypeStruct((NC, N, 16), jnp.float32),
      scratch_types=[
          pltpu.VMEM_SHARED((N, 16), jnp.float32),
          pltpu.VMEM((NCHUNK, CH), jnp.int32),
          pltpu.VMEM((CH, 16), jnp.float32),
          pltpu.VMEM((RCH, 16), jnp.float32),
      ])
  def k(edge_hbm, out_hbm, acc, idx_v, ones_v, stage_v):
    c = lax.axis_index("c")
    s = lax.axis_index("s")
    wid = c * NS + s
    one = jnp.ones((16,), jnp.float32)
    for r in range(CH):
      ones_v[r, pl.ds(0, 16)] = one
    zero = jnp.zeros((16,), jnp.float32)
    for r in range(RCH):
      stage_v[r, pl.ds(0, 16)] = zero
    pltpu.sync_copy(edge_hbm.at[1, pl.ds(wid * NCHUNK, NCHUNK)], idx_v)
    for r in range(NR):
      pltpu.sync_copy(stage_v, acc.at[pl.ds(s * RPT + r * RCH, RCH)])
    plsc.subcore_barrier()

    def body(j, carry):
      pltpu.sync_copy(ones_v, acc.at[idx_v.at[j]], add=True)
      return carry

    lax.fori_loop(0, NCHUNK, body, 0)
    plsc.subcore_barrier()
    for r in range(NR):
      rr = s * RPT + r * RCH
      pltpu.sync_copy(acc.at[pl.ds(rr, RCH)], stage_v)
      pltpu.sync_copy(stage_v, out_hbm.at[c, pl.ds(rr, RCH)])

  return k


@functools.lru_cache(maxsize=None)
def _agg_call(N, D, E):
  EW = E // NW
  CH = 80
  NCHUNK = EW // CH
  RPT = N // NS
  RCH = 25
  NR = RPT // RCH

  @functools.partial(
      pl.kernel, mesh=_sc_mesh(), compiler_params=_SC_PARAMS,
      out_type=jax.ShapeDtypeStruct((NC, N, D), jnp.float32),
      scratch_types=[
          pltpu.VMEM_SHARED((N, D), jnp.float32),
          pltpu.VMEM((NCHUNK, CH), jnp.int32),
          pltpu.VMEM((NCHUNK, CH), jnp.int32),
          pltpu.VMEM((CH, D), jnp.float32),
          pltpu.VMEM((CH, D), jnp.float32),
          pltpu.VMEM((RCH, D), jnp.float32),
          pltpu.SemaphoreType.DMA,
          pltpu.SemaphoreType.DMA,
      ])
  def k(y_hbm, edge_hbm, out_hbm,
        acc, si_v, di_v, rows_a, rows_b, stage_v, sem_a, sem_b):
    c = lax.axis_index("c")
    s = lax.axis_index("s")
    wid = c * NS + s
    pltpu.sync_copy(edge_hbm.at[0, pl.ds(wid * NCHUNK, NCHUNK)], si_v)
    pltpu.sync_copy(edge_hbm.at[1, pl.ds(wid * NCHUNK, NCHUNK)], di_v)

    @pl.when(c == 0)
    def _():
      for r in range(NR):
        rr = s * RPT + r * RCH
        pltpu.sync_copy(y_hbm.at[pl.ds(rr, RCH)], stage_v)
        pltpu.sync_copy(stage_v, acc.at[pl.ds(rr, RCH)])

    @pl.when(c == 1)
    def _():
      zero = jnp.zeros((16,), jnp.float32)
      for r in range(RCH):
        for l in range(D // 16):
          stage_v[r, pl.ds(l * 16, 16)] = zero
      for r in range(NR):
        pltpu.sync_copy(stage_v, acc.at[pl.ds(s * RPT + r * RCH, RCH)])

    plsc.subcore_barrier()

    def wait_g(rows, sem):
      pltpu.make_async_copy(y_hbm.at[si_v.at[0]], rows, sem).wait()

    pltpu.async_copy(y_hbm.at[si_v.at[0]], rows_a, sem_a)

    def body(j2, carry):
      c0 = j2 * 2
      pltpu.async_copy(y_hbm.at[si_v.at[c0 + 1]], rows_b, sem_b)
      wait_g(rows_a, sem_a)
      pltpu.sync_copy(rows_a, acc.at[di_v.at[c0]], add=True)
      pltpu.async_copy(y_hbm.at[si_v.at[c0 + 2]], rows_a, sem_a)
      wait_g(rows_b, sem_b)
      pltpu.sync_copy(rows_b, acc.at[di_v.at[c0 + 1]], add=True)
      return carry

    lax.fori_loop(0, (NCHUNK - 1) // 2, body, 0)
    wait_g(rows_a, sem_a)
    pltpu.sync_copy(rows_a, acc.at[di_v.at[NCHUNK - 1]], add=True)
    plsc.subcore_barrier()
    for r in range(NR):
      rr = s * RPT + r * RCH
      pltpu.sync_copy(acc.at[pl.ds(rr, RCH)], stage_v)
      pltpu.sync_copy(stage_v, out_hbm.at[c, pl.ds(rr, RCH)])

  return k


def _dis(d_ref):
  return lax.rsqrt(d_ref[0, :, :1] + d_ref[1, :, :1] + 1.0)


def _tc_in(x, W0, degp):
  N, D = x.shape
  H = W0.shape[1]
  B = 1000

  def body(x_ref, w_ref, d_ref, o_ref):
    y = jnp.dot(x_ref[...], w_ref[...], preferred_element_type=jnp.float32)
    o_ref[...] = y * _dis(d_ref)

  return pl.pallas_call(
      body, grid=(N // B,),
      in_specs=[
          pl.BlockSpec((B, D), lambda i: (i, 0)),
          pl.BlockSpec((D, H), lambda i: (0, 0)),
          pl.BlockSpec((2, B, 16), lambda i: (0, i, 0)),
      ],
      out_specs=pl.BlockSpec((B, H), lambda i: (i, 0)),
      out_shape=jax.ShapeDtypeStruct((N, H), jnp.float32),
  )(x, W0, degp)


def _tc_mid(accp, degp, b0, W1):
  _, N, H = accp.shape
  B = 1000

  def body(a_ref, d_ref, b_ref, w_ref, o_ref):
    dis = _dis(d_ref)
    h = (a_ref[0] + a_ref[1]) * dis + b_ref[...]
    h = jnp.maximum(h, 0.0)
    o_ref[...] = jnp.dot(
        h, w_ref[...], preferred_element_type=jnp.float32) * dis

  return pl.pallas_call(
      body, grid=(N // B,),
      in_specs=[
          pl.BlockSpec((2, B, H), lambda i: (0, i, 0)),
          pl.BlockSpec((2, B, 16), lambda i: (0, i, 0)),
          pl.BlockSpec((1, H), lambda i: (0, 0)),
          pl.BlockSpec((H, H), lambda i: (0, 0)),
      ],
      out_specs=pl.BlockSpec((B, H), lambda i: (i, 0)),
      out_shape=jax.ShapeDtypeStruct((N, H), jnp.float32),
  )(accp, degp, b0, W1)


def _tc_out(accp, degp, b1, batch3d, Wout, bout, G):
  _, N, H = accp.shape
  OUT = Wout.shape[1]
  B = 1000
  nblk = N // B

  def body(a_ref, d_ref, b_ref, bt_ref, wo_ref,
           bo_ref, o_ref, pooled):
    i = pl.program_id(0)
    dis = _dis(d_ref)
    h = (a_ref[0] + a_ref[1]) * dis + b_ref[...]
    h = jnp.maximum(h, 0.0)
    seg = bt_ref[...].reshape(1, B)
    oh = (lax.broadcasted_iota(jnp.int32, (G, B), 0)
          == jnp.broadcast_to(seg, (G, B))).astype(jnp.float32)

    @pl.when(i == 0)
    def _():
      pooled[...] = jnp.zeros_like(pooled)

    pooled[...] += jnp.dot(oh, h, preferred_element_type=jnp.float32)

    @pl.when(i == nblk - 1)
    def _():
      o_ref[...] = jnp.dot(
          pooled[...], wo_ref[...],
          preferred_element_type=jnp.float32) + bo_ref[...]

  return pl.pallas_call(
      body, grid=(nblk,),
      in_specs=[
          pl.BlockSpec((2, B, H), lambda i: (0, i, 0)),
          pl.BlockSpec((2, B, 16), lambda i: (0, i, 0)),
          pl.BlockSpec((1, H), lambda i: (0, 0)),
          pl.BlockSpec((1, 1, B), lambda i: (i, 0, 0)),
          pl.BlockSpec((H, OUT), lambda i: (0, 0)),
          pl.BlockSpec((1, OUT), lambda i: (0, 0)),
      ],
      out_specs=pl.BlockSpec((G, OUT), lambda i: (0, 0)),
      out_shape=jax.ShapeDtypeStruct((G, OUT), jnp.float32),
      scratch_shapes=[pltpu.VMEM((G, OUT), jnp.float32)],
  )(accp, degp, b1, batch3d, Wout, bout)


def kernel(x, edge_index, batch, W0, b0, W1, b1, Wout, bout):
  N, D = x.shape
  E = edge_index.shape[1]
  H = W0.shape[1]
  G = 64
  B = 1000

  CH = 80
  NCHUNK = E // NW // CH
  edge3 = edge_index.reshape(2, NW * NCHUNK, CH)

  degp = _deg_call(N, E)(edge3)
  y0 = _tc_in(x, W0, degp)
  acc = _agg_call(N, D, E)(y0, edge3)
  y1 = _tc_mid(acc, degp, b0.reshape(1, H), W1)
  acc = _agg_call(N, H, E)(y1, edge3)
  return _tc_out(acc, degp, b1.reshape(1, H),
                 batch.reshape(N // B, 1, B), Wout,
                 bout.reshape(1, Wout.shape[1]), G)


# speedup vs baseline: 1.1091x; 1.1091x over previous
import functools

import jax
import jax.numpy as jnp
from jax import lax
from jax.experimental import pallas as pl
from jax.experimental.pallas import tpu as pltpu
from jax.experimental.pallas import tpu_sc as plsc

NC = 2
NS = 16
NW = NC * NS


def _sc_mesh():
  return plsc.VectorSubcoreMesh(
      core_axis_name="c", subcore_axis_name="s", num_cores=NC,
      num_subcores=NS)


_SC_PARAMS = pltpu.CompilerParams(use_tc_tiling_on_sc=False)


@functools.lru_cache(maxsize=None)
def _deg_call(N, E):
  EW = E // NW
  CH = 80
  NCHUNK = EW // CH
  RPT = N // NS
  RCH = 125
  NR = RPT // RCH

  @functools.partial(
      pl.kernel, mesh=_sc_mesh(), compiler_params=_SC_PARAMS,
      out_type=jax.ShapeDtypeStruct((NC, N, 16), jnp.float32),
      scratch_types=[
          pltpu.VMEM_SHARED((N, 16), jnp.float32),
          pltpu.VMEM((NCHUNK, CH), jnp.int32),
          pltpu.VMEM((CH, 16), jnp.float32),
          pltpu.VMEM((RCH, 16), jnp.float32),
      ])
  def k(edge_hbm, out_hbm, acc, idx_v, ones_v, stage_v):
    c = lax.axis_index("c")
    s = lax.axis_index("s")
    wid = c * NS + s
    one = jnp.ones((16,), jnp.float32)
    for r in range(CH):
      ones_v[r, pl.ds(0, 16)] = one
    zero = jnp.zeros((16,), jnp.float32)
    for r in range(RCH):
      stage_v[r, pl.ds(0, 16)] = zero
    pltpu.sync_copy(edge_hbm.at[1, pl.ds(wid * NCHUNK, NCHUNK)], idx_v)
    for r in range(NR):
      pltpu.sync_copy(stage_v, acc.at[pl.ds(s * RPT + r * RCH, RCH)])
    plsc.subcore_barrier()

    def body(j, carry):
      pltpu.sync_copy(ones_v, acc.at[idx_v.at[j]], add=True)
      return carry

    lax.fori_loop(0, NCHUNK, body, 0)
    plsc.subcore_barrier()
    for r in range(NR):
      rr = s * RPT + r * RCH
      pltpu.sync_copy(acc.at[pl.ds(rr, RCH)], stage_v)
      pltpu.sync_copy(stage_v, out_hbm.at[c, pl.ds(rr, RCH)])

  return k


@functools.lru_cache(maxsize=None)
def _agg_call(N, D, E):
  EW = E // NW
  CH = 80
  NCHUNK = EW // CH
  RPT = N // NS
  RCH = 25
  NR = RPT // RCH

  @functools.partial(
      pl.kernel, mesh=_sc_mesh(), compiler_params=_SC_PARAMS,
      out_type=jax.ShapeDtypeStruct((NC, N, D), jnp.float32),
      scratch_types=[
          pltpu.VMEM_SHARED((N, D), jnp.float32),
          pltpu.VMEM((NCHUNK, CH), jnp.int32),
          pltpu.VMEM((NCHUNK, CH), jnp.int32),
          pltpu.VMEM((CH, D), jnp.float32),
          pltpu.VMEM((CH, D), jnp.float32),
          pltpu.VMEM((RCH, D), jnp.float32),
          pltpu.SemaphoreType.DMA,
          pltpu.SemaphoreType.DMA,
      ])
  def k(y_hbm, edge_hbm, out_hbm,
        acc, si_v, di_v, rows_a, rows_b, stage_v, sem_a, sem_b):
    c = lax.axis_index("c")
    s = lax.axis_index("s")
    wid = c * NS + s
    pltpu.sync_copy(edge_hbm.at[0, pl.ds(wid * NCHUNK, NCHUNK)], si_v)
    pltpu.sync_copy(edge_hbm.at[1, pl.ds(wid * NCHUNK, NCHUNK)], di_v)
    zero = jnp.zeros((16,), jnp.float32)
    for r in range(RCH):
      for l in range(D // 16):
        stage_v[r, pl.ds(l * 16, 16)] = zero
    for r in range(NR):
      pltpu.sync_copy(stage_v, acc.at[pl.ds(s * RPT + r * RCH, RCH)])
    plsc.subcore_barrier()

    def wait_g(rows, sem):
      pltpu.make_async_copy(y_hbm.at[si_v.at[0]], rows, sem).wait()

    pltpu.async_copy(y_hbm.at[si_v.at[0]], rows_a, sem_a)

    def body(j2, carry):
      c0 = j2 * 2
      pltpu.async_copy(y_hbm.at[si_v.at[c0 + 1]], rows_b, sem_b)
      wait_g(rows_a, sem_a)
      pltpu.sync_copy(rows_a, acc.at[di_v.at[c0]], add=True)
      pltpu.async_copy(y_hbm.at[si_v.at[c0 + 2]], rows_a, sem_a)
      wait_g(rows_b, sem_b)
      pltpu.sync_copy(rows_b, acc.at[di_v.at[c0 + 1]], add=True)
      return carry

    lax.fori_loop(0, (NCHUNK - 1) // 2, body, 0)
    wait_g(rows_a, sem_a)
    pltpu.sync_copy(rows_a, acc.at[di_v.at[NCHUNK - 1]], add=True)
    plsc.subcore_barrier()
    for r in range(NR):
      rr = s * RPT + r * RCH
      pltpu.sync_copy(acc.at[pl.ds(rr, RCH)], stage_v)
      pltpu.sync_copy(stage_v, out_hbm.at[c, pl.ds(rr, RCH)])

  return k


def _dis(d_ref):
  return lax.rsqrt(d_ref[0, :, :1] + d_ref[1, :, :1] + 1.0)


def _tc_in(x, W0, degp):
  N, D = x.shape
  H = W0.shape[1]
  B = 1000

  def body(x_ref, w_ref, d_ref, o_ref):
    y = jnp.dot(x_ref[...], w_ref[...], preferred_element_type=jnp.float32)
    o_ref[...] = y * _dis(d_ref)

  return pl.pallas_call(
      body, grid=(N // B,),
      in_specs=[
          pl.BlockSpec((B, D), lambda i: (i, 0)),
          pl.BlockSpec((D, H), lambda i: (0, 0)),
          pl.BlockSpec((2, B, 16), lambda i: (0, i, 0)),
      ],
      out_specs=pl.BlockSpec((B, H), lambda i: (i, 0)),
      out_shape=jax.ShapeDtypeStruct((N, H), jnp.float32),
  )(x, W0, degp)


def _tc_mid(accp, y0, degp, b0, W1):
  N, H = y0.shape
  B = 1000

  def body(a_ref, y_ref, d_ref, b_ref, w_ref, o_ref):
    dis = _dis(d_ref)
    h = (a_ref[0] + a_ref[1] + y_ref[...]) * dis + b_ref[...]
    h = jnp.maximum(h, 0.0)
    o_ref[...] = jnp.dot(
        h, w_ref[...], preferred_element_type=jnp.float32) * dis

  return pl.pallas_call(
      body, grid=(N // B,),
      in_specs=[
          pl.BlockSpec((2, B, H), lambda i: (0, i, 0)),
          pl.BlockSpec((B, H), lambda i: (i, 0)),
          pl.BlockSpec((2, B, 16), lambda i: (0, i, 0)),
          pl.BlockSpec((1, H), lambda i: (0, 0)),
          pl.BlockSpec((H, H), lambda i: (0, 0)),
      ],
      out_specs=pl.BlockSpec((B, H), lambda i: (i, 0)),
      out_shape=jax.ShapeDtypeStruct((N, H), jnp.float32),
  )(accp, y0, degp, b0, W1)


def _tc_out(accp, y1, degp, b1, batch3d, Wout, bout, G):
  N, H = y1.shape
  OUT = Wout.shape[1]
  B = 1000
  nblk = N // B

  def body(a_ref, y_ref, d_ref, b_ref, bt_ref, wo_ref,
           bo_ref, o_ref, pooled):
    i = pl.program_id(0)
    dis = _dis(d_ref)
    h = (a_ref[0] + a_ref[1] + y_ref[...]) * dis + b_ref[...]
    h = jnp.maximum(h, 0.0)
    seg = bt_ref[...].reshape(1, B)
    oh = (lax.broadcasted_iota(jnp.int32, (G, B), 0)
          == jnp.broadcast_to(seg, (G, B))).astype(jnp.float32)

    @pl.when(i == 0)
    def _():
      pooled[...] = jnp.zeros_like(pooled)

    pooled[...] += jnp.dot(oh, h, preferred_element_type=jnp.float32)

    @pl.when(i == nblk - 1)
    def _():
      o_ref[...] = jnp.dot(
          pooled[...], wo_ref[...],
          preferred_element_type=jnp.float32) + bo_ref[...]

  return pl.pallas_call(
      body, grid=(nblk,),
      in_specs=[
          pl.BlockSpec((2, B, H), lambda i: (0, i, 0)),
          pl.BlockSpec((B, H), lambda i: (i, 0)),
          pl.BlockSpec((2, B, 16), lambda i: (0, i, 0)),
          pl.BlockSpec((1, H), lambda i: (0, 0)),
          pl.BlockSpec((1, 1, B), lambda i: (i, 0, 0)),
          pl.BlockSpec((H, OUT), lambda i: (0, 0)),
          pl.BlockSpec((1, OUT), lambda i: (0, 0)),
      ],
      out_specs=pl.BlockSpec((G, OUT), lambda i: (0, 0)),
      out_shape=jax.ShapeDtypeStruct((G, OUT), jnp.float32),
      scratch_shapes=[pltpu.VMEM((G, OUT), jnp.float32)],
  )(accp, y1, degp, b1, batch3d, Wout, bout)


def kernel(x, edge_index, batch, W0, b0, W1, b1, Wout, bout):
  N, D = x.shape
  E = edge_index.shape[1]
  H = W0.shape[1]
  G = 64
  B = 1000

  CH = 80
  NCHUNK = E // NW // CH
  edge3 = edge_index.reshape(2, NW * NCHUNK, CH)

  degp = _deg_call(N, E)(edge3)
  y0 = _tc_in(x, W0, degp)
  acc = _agg_call(N, D, E)(y0, edge3)
  y1 = _tc_mid(acc, y0, degp, b0.reshape(1, H), W1)
  acc = _agg_call(N, H, E)(y1, edge3)
  return _tc_out(acc, y1, degp, b1.reshape(1, H),
                 batch.reshape(N // B, 1, B), Wout,
                 bout.reshape(1, Wout.shape[1]), G)


# speedup vs baseline: 1.1386x; 1.0267x over previous
import functools

import jax
import jax.numpy as jnp
from jax import lax
from jax.experimental import pallas as pl
from jax.experimental.pallas import tpu as pltpu
from jax.experimental.pallas import tpu_sc as plsc

NC = 2
NS = 16
NW = NC * NS


def _sc_mesh():
  return plsc.VectorSubcoreMesh(
      core_axis_name="c", subcore_axis_name="s", num_cores=NC,
      num_subcores=NS)


_SC_PARAMS = pltpu.CompilerParams(use_tc_tiling_on_sc=False)


@functools.lru_cache(maxsize=None)
def _deg_call(N, E):
  EW = E // NW
  CH = 100
  NCHUNK = EW // CH
  RPT = N // NS
  RCH = 125
  NR = RPT // RCH

  @functools.partial(
      pl.kernel, mesh=_sc_mesh(), compiler_params=_SC_PARAMS,
      out_type=jax.ShapeDtypeStruct((NC, N, 16), jnp.float32),
      scratch_types=[
          pltpu.VMEM_SHARED((N, 16), jnp.float32),
          pltpu.VMEM((NCHUNK, CH), jnp.int32),
          pltpu.VMEM((CH, 16), jnp.float32),
          pltpu.VMEM((RCH, 16), jnp.float32),
          pltpu.SemaphoreType.DMA,
          pltpu.SemaphoreType.DMA,
      ])
  def k(edge_hbm, out_hbm, acc, idx_v, ones_v, stage_v, sem_a, sem_b):
    c = lax.axis_index("c")
    s = lax.axis_index("s")
    wid = c * NS + s
    one = jnp.ones((16,), jnp.float32)
    for r in range(CH):
      ones_v[r, pl.ds(0, 16)] = one
    zero = jnp.zeros((16,), jnp.float32)
    for r in range(RCH):
      stage_v[r, pl.ds(0, 16)] = zero
    pltpu.sync_copy(edge_hbm.at[1, pl.ds(wid * NCHUNK, NCHUNK)], idx_v)
    for r in range(NR):
      pltpu.sync_copy(stage_v, acc.at[pl.ds(s * RPT + r * RCH, RCH)])
    plsc.subcore_barrier()

    def wait_s(sem):
      pltpu.make_async_copy(ones_v, acc.at[idx_v.at[0]], sem).wait()

    pltpu.async_copy(ones_v, acc.at[idx_v.at[0]], sem_a, add=True)

    def body(j2, carry):
      c0 = j2 * 2
      pltpu.async_copy(ones_v, acc.at[idx_v.at[c0 + 1]], sem_b, add=True)
      wait_s(sem_a)

      @pl.when(c0 + 2 < NCHUNK)
      def _():
        pltpu.async_copy(ones_v, acc.at[idx_v.at[c0 + 2]], sem_a, add=True)

      wait_s(sem_b)
      return carry

    lax.fori_loop(0, NCHUNK // 2, body, 0)
    if NCHUNK % 2:
      wait_s(sem_a)
    plsc.subcore_barrier()
    for r in range(NR):
      rr = s * RPT + r * RCH
      pltpu.sync_copy(acc.at[pl.ds(rr, RCH)], stage_v)
      pltpu.sync_copy(stage_v, out_hbm.at[c, pl.ds(rr, RCH)])

  return k


@functools.lru_cache(maxsize=None)
def _agg_call(N, D, E):
  EW = E // NW
  CH = 100
  NCHUNK = EW // CH
  RPT = N // NS
  RCH = 25
  NR = RPT // RCH

  @functools.partial(
      pl.kernel, mesh=_sc_mesh(), compiler_params=_SC_PARAMS,
      out_type=jax.ShapeDtypeStruct((NC, N, D), jnp.float32),
      scratch_types=[
          pltpu.VMEM_SHARED((N, D), jnp.float32),
          pltpu.VMEM((NCHUNK, CH), jnp.int32),
          pltpu.VMEM((NCHUNK, CH), jnp.int32),
          pltpu.VMEM((CH, D), jnp.float32),
          pltpu.VMEM((CH, D), jnp.float32),
          pltpu.VMEM((RCH, D), jnp.float32),
          pltpu.SemaphoreType.DMA,
          pltpu.SemaphoreType.DMA,
      ])
  def k(y_hbm, edge_hbm, out_hbm,
        acc, si_v, di_v, rows_a, rows_b, stage_v, sem_a, sem_b):
    c = lax.axis_index("c")
    s = lax.axis_index("s")
    wid = c * NS + s
    pltpu.sync_copy(edge_hbm.at[0, pl.ds(wid * NCHUNK, NCHUNK)], si_v)
    pltpu.sync_copy(edge_hbm.at[1, pl.ds(wid * NCHUNK, NCHUNK)], di_v)
    zero = jnp.zeros((16,), jnp.float32)
    for r in range(RCH):
      for l in range(D // 16):
        stage_v[r, pl.ds(l * 16, 16)] = zero
    for r in range(NR):
      pltpu.sync_copy(stage_v, acc.at[pl.ds(s * RPT + r * RCH, RCH)])
    plsc.subcore_barrier()

    def wait_g(rows, sem):
      pltpu.make_async_copy(y_hbm.at[si_v.at[0]], rows, sem).wait()

    pltpu.async_copy(y_hbm.at[si_v.at[0]], rows_a, sem_a)

    def body(j2, carry):
      c0 = j2 * 2
      pltpu.async_copy(y_hbm.at[si_v.at[c0 + 1]], rows_b, sem_b)
      wait_g(rows_a, sem_a)
      pltpu.sync_copy(rows_a, acc.at[di_v.at[c0]], add=True)

      @pl.when(c0 + 2 < NCHUNK)
      def _():
        pltpu.async_copy(y_hbm.at[si_v.at[c0 + 2]], rows_a, sem_a)

      wait_g(rows_b, sem_b)
      pltpu.sync_copy(rows_b, acc.at[di_v.at[c0 + 1]], add=True)
      return carry

    lax.fori_loop(0, NCHUNK // 2, body, 0)
    if NCHUNK % 2:
      wait_g(rows_a, sem_a)
      pltpu.sync_copy(rows_a, acc.at[di_v.at[NCHUNK - 1]], add=True)
    plsc.subcore_barrier()
    for r in range(NR):
      rr = s * RPT + r * RCH
      pltpu.sync_copy(acc.at[pl.ds(rr, RCH)], stage_v)
      pltpu.sync_copy(stage_v, out_hbm.at[c, pl.ds(rr, RCH)])

  return k


def _dis(d_ref):
  return lax.rsqrt(d_ref[0, :, :1] + d_ref[1, :, :1] + 1.0)


def _tc_in(x, W0, degp):
  N, D = x.shape
  H = W0.shape[1]
  B = 1000

  def body(x_ref, w_ref, d_ref, o_ref):
    y = jnp.dot(x_ref[...], w_ref[...], preferred_element_type=jnp.float32)
    o_ref[...] = y * _dis(d_ref)

  return pl.pallas_call(
      body, grid=(N // B,),
      in_specs=[
          pl.BlockSpec((B, D), lambda i: (i, 0)),
          pl.BlockSpec((D, H), lambda i: (0, 0)),
          pl.BlockSpec((2, B, 16), lambda i: (0, i, 0)),
      ],
      out_specs=pl.BlockSpec((B, H), lambda i: (i, 0)),
      out_shape=jax.ShapeDtypeStruct((N, H), jnp.float32),
  )(x, W0, degp)


def _tc_mid(accp, y0, degp, b0, W1):
  N, H = y0.shape
  B = 1000

  def body(a_ref, y_ref, d_ref, b_ref, w_ref, o_ref):
    dis = _dis(d_ref)
    h = (a_ref[0] + a_ref[1] + y_ref[...]) * dis + b_ref[...]
    h = jnp.maximum(h, 0.0)
    o_ref[...] = jnp.dot(
        h, w_ref[...], preferred_element_type=jnp.float32) * dis

  return pl.pallas_call(
      body, grid=(N // B,),
      in_specs=[
          pl.BlockSpec((2, B, H), lambda i: (0, i, 0)),
          pl.BlockSpec((B, H), lambda i: (i, 0)),
          pl.BlockSpec((2, B, 16), lambda i: (0, i, 0)),
          pl.BlockSpec((1, H), lambda i: (0, 0)),
          pl.BlockSpec((H, H), lambda i: (0, 0)),
      ],
      out_specs=pl.BlockSpec((B, H), lambda i: (i, 0)),
      out_shape=jax.ShapeDtypeStruct((N, H), jnp.float32),
  )(accp, y0, degp, b0, W1)


def _tc_out(accp, y1, degp, b1, batch3d, Wout, bout, G):
  N, H = y1.shape
  OUT = Wout.shape[1]
  B = 1000
  nblk = N // B

  def body(a_ref, y_ref, d_ref, b_ref, bt_ref, wo_ref,
           bo_ref, o_ref, pooled):
    i = pl.program_id(0)
    dis = _dis(d_ref)
    h = (a_ref[0] + a_ref[1] + y_ref[...]) * dis + b_ref[...]
    h = jnp.maximum(h, 0.0)
    seg = bt_ref[...].reshape(1, B)
    oh = (lax.broadcasted_iota(jnp.int32, (G, B), 0)
          == jnp.broadcast_to(seg, (G, B))).astype(jnp.float32)

    @pl.when(i == 0)
    def _():
      pooled[...] = jnp.zeros_like(pooled)

    pooled[...] += jnp.dot(oh, h, preferred_element_type=jnp.float32)

    @pl.when(i == nblk - 1)
    def _():
      o_ref[...] = jnp.dot(
          pooled[...], wo_ref[...],
          preferred_element_type=jnp.float32) + bo_ref[...]

  return pl.pallas_call(
      body, grid=(nblk,),
      in_specs=[
          pl.BlockSpec((2, B, H), lambda i: (0, i, 0)),
          pl.BlockSpec((B, H), lambda i: (i, 0)),
          pl.BlockSpec((2, B, 16), lambda i: (0, i, 0)),
          pl.BlockSpec((1, H), lambda i: (0, 0)),
          pl.BlockSpec((1, 1, B), lambda i: (i, 0, 0)),
          pl.BlockSpec((H, OUT), lambda i: (0, 0)),
          pl.BlockSpec((1, OUT), lambda i: (0, 0)),
      ],
      out_specs=pl.BlockSpec((G, OUT), lambda i: (0, 0)),
      out_shape=jax.ShapeDtypeStruct((G, OUT), jnp.float32),
      scratch_shapes=[pltpu.VMEM((G, OUT), jnp.float32)],
  )(accp, y1, degp, b1, batch3d, Wout, bout)


def kernel(x, edge_index, batch, W0, b0, W1, b1, Wout, bout):
  N, D = x.shape
  E = edge_index.shape[1]
  H = W0.shape[1]
  G = 64
  B = 1000

  CH = 100
  NCHUNK = E // NW // CH
  edge3 = edge_index.reshape(2, NW * NCHUNK, CH)

  degp = _deg_call(N, E)(edge3)
  y0 = _tc_in(x, W0, degp)
  acc = _agg_call(N, D, E)(y0, edge3)
  y1 = _tc_mid(acc, y0, degp, b0.reshape(1, H), W1)
  acc = _agg_call(N, H, E)(y1, edge3)
  return _tc_out(acc, y1, degp, b1.reshape(1, H),
                 batch.reshape(N // B, 1, B), Wout,
                 bout.reshape(1, Wout.shape[1]), G)


# speedup vs baseline: 1.1659x; 1.0239x over previous
import functools

import jax
import jax.numpy as jnp
from jax import lax
from jax.experimental import pallas as pl
from jax.experimental.pallas import tpu as pltpu
from jax.experimental.pallas import tpu_sc as plsc

NC = 2
NS = 16
NW = NC * NS


def _sc_mesh():
  return plsc.VectorSubcoreMesh(
      core_axis_name="c", subcore_axis_name="s", num_cores=NC,
      num_subcores=NS)


_SC_PARAMS = pltpu.CompilerParams(use_tc_tiling_on_sc=False)


@functools.lru_cache(maxsize=None)
def _deg_call(N, E):
  EW = E // NW
  CH = 100
  NCHUNK = EW // CH
  RPT = N // NS
  RCH = 125
  NR = RPT // RCH

  @functools.partial(
      pl.kernel, mesh=_sc_mesh(), compiler_params=_SC_PARAMS,
      out_type=jax.ShapeDtypeStruct((NC, N, 16), jnp.float32),
      scratch_types=[
          pltpu.VMEM_SHARED((N, 16), jnp.float32),
          pltpu.VMEM((NCHUNK, CH), jnp.int32),
          pltpu.VMEM((CH, 16), jnp.float32),
          pltpu.VMEM((RCH, 16), jnp.float32),
          pltpu.SemaphoreType.DMA,
          pltpu.SemaphoreType.DMA,
      ])
  def k(edge_hbm, out_hbm, acc, idx_v, ones_v, stage_v, sem_a, sem_b):
    c = lax.axis_index("c")
    s = lax.axis_index("s")
    wid = c * NS + s
    one = jnp.ones((16,), jnp.float32)
    for r in range(CH):
      ones_v[r, pl.ds(0, 16)] = one
    zero = jnp.zeros((16,), jnp.float32)
    for r in range(RCH):
      stage_v[r, pl.ds(0, 16)] = zero
    pltpu.sync_copy(edge_hbm.at[1, pl.ds(wid * NCHUNK, NCHUNK)], idx_v)
    for r in range(NR):
      pltpu.sync_copy(stage_v, acc.at[pl.ds(s * RPT + r * RCH, RCH)])
    plsc.subcore_barrier()

    def wait_s(sem):
      pltpu.make_async_copy(ones_v, acc.at[idx_v.at[0]], sem).wait()

    pltpu.async_copy(ones_v, acc.at[idx_v.at[0]], sem_a, add=True)

    def body(j2, carry):
      c0 = j2 * 2
      pltpu.async_copy(ones_v, acc.at[idx_v.at[c0 + 1]], sem_b, add=True)
      wait_s(sem_a)

      @pl.when(c0 + 2 < NCHUNK)
      def _():
        pltpu.async_copy(ones_v, acc.at[idx_v.at[c0 + 2]], sem_a, add=True)

      wait_s(sem_b)
      return carry

    lax.fori_loop(0, NCHUNK // 2, body, 0)
    if NCHUNK % 2:
      wait_s(sem_a)
    plsc.subcore_barrier()
    for r in range(NR):
      rr = s * RPT + r * RCH
      pltpu.sync_copy(acc.at[pl.ds(rr, RCH)], stage_v)
      pltpu.sync_copy(stage_v, out_hbm.at[c, pl.ds(rr, RCH)])

  return k


@functools.lru_cache(maxsize=None)
def _agg_call(N, D, E):
  EW = E // NW
  CH = 100
  NCHUNK = EW // CH
  RPT = N // NS
  RCH = 25
  NR = RPT // RCH

  @functools.partial(
      pl.kernel, mesh=_sc_mesh(), compiler_params=_SC_PARAMS,
      out_type=jax.ShapeDtypeStruct((NC, N, D), jnp.float32),
      scratch_types=[
          pltpu.VMEM_SHARED((N, D), jnp.float32),
          pltpu.VMEM((NCHUNK, CH), jnp.int32),
          pltpu.VMEM((NCHUNK, CH), jnp.int32),
          pltpu.VMEM((CH, D), jnp.float32),
          pltpu.VMEM((CH, D), jnp.float32),
          pltpu.VMEM((RCH, D), jnp.float32),
          pltpu.SemaphoreType.DMA,
          pltpu.SemaphoreType.DMA,
      ])
  def k(y_hbm, edge_hbm, out_hbm,
        acc, si_v, di_v, rows_a, rows_b, stage_v, sem_a, sem_b):
    c = lax.axis_index("c")
    s = lax.axis_index("s")
    wid = c * NS + s
    pltpu.sync_copy(edge_hbm.at[0, pl.ds(wid * NCHUNK, NCHUNK)], si_v)
    pltpu.sync_copy(edge_hbm.at[1, pl.ds(wid * NCHUNK, NCHUNK)], di_v)
    zero = jnp.zeros((16,), jnp.float32)
    for r in range(RCH):
      for l in range(D // 16):
        stage_v[r, pl.ds(l * 16, 16)] = zero
    for r in range(NR):
      pltpu.sync_copy(stage_v, acc.at[pl.ds(s * RPT + r * RCH, RCH)])
    plsc.subcore_barrier()

    def wait_g(rows, sem):
      pltpu.make_async_copy(y_hbm.at[si_v.at[0]], rows, sem).wait()

    pltpu.async_copy(y_hbm.at[si_v.at[0]], rows_a, sem_a)

    def body(j2, carry):
      c0 = j2 * 2
      pltpu.async_copy(y_hbm.at[si_v.at[c0 + 1]], rows_b, sem_b)
      wait_g(rows_a, sem_a)
      pltpu.sync_copy(rows_a, acc.at[di_v.at[c0]], add=True)

      @pl.when(c0 + 2 < NCHUNK)
      def _():
        pltpu.async_copy(y_hbm.at[si_v.at[c0 + 2]], rows_a, sem_a)

      wait_g(rows_b, sem_b)
      pltpu.sync_copy(rows_b, acc.at[di_v.at[c0 + 1]], add=True)
      return carry

    lax.fori_loop(0, NCHUNK // 2, body, 0)
    if NCHUNK % 2:
      wait_g(rows_a, sem_a)
      pltpu.sync_copy(rows_a, acc.at[di_v.at[NCHUNK - 1]], add=True)
    plsc.subcore_barrier()
    for r in range(NR):
      rr = s * RPT + r * RCH
      pltpu.sync_copy(acc.at[pl.ds(rr, RCH)], stage_v)
      pltpu.sync_copy(stage_v, out_hbm.at[c, pl.ds(rr, RCH)])

  return k


def _dis(d_ref):
  return lax.rsqrt(d_ref[0, :, :1] + d_ref[1, :, :1] + 1.0)


def _tc_in(x, W0, degp):
  N, D = x.shape
  H = W0.shape[1]
  B = 2000

  def body(x_ref, w_ref, d_ref, o_ref):
    y = jnp.dot(x_ref[...], w_ref[...], preferred_element_type=jnp.float32)
    o_ref[...] = y * _dis(d_ref)

  return pl.pallas_call(
      body, grid=(N // B,),
      in_specs=[
          pl.BlockSpec((B, D), lambda i: (i, 0)),
          pl.BlockSpec((D, H), lambda i: (0, 0)),
          pl.BlockSpec((2, B, 16), lambda i: (0, i, 0)),
      ],
      out_specs=pl.BlockSpec((B, H), lambda i: (i, 0)),
      out_shape=jax.ShapeDtypeStruct((N, H), jnp.float32),
  )(x, W0, degp)


def _tc_mid(accp, y0, degp, b0, W1):
  N, H = y0.shape
  B = 2000

  def body(a_ref, y_ref, d_ref, b_ref, w_ref, o_ref):
    dis = _dis(d_ref)
    h = (a_ref[0] + a_ref[1] + y_ref[...]) * dis + b_ref[...]
    h = jnp.maximum(h, 0.0)
    o_ref[...] = jnp.dot(
        h, w_ref[...], preferred_element_type=jnp.float32) * dis

  return pl.pallas_call(
      body, grid=(N // B,),
      in_specs=[
          pl.BlockSpec((2, B, H), lambda i: (0, i, 0)),
          pl.BlockSpec((B, H), lambda i: (i, 0)),
          pl.BlockSpec((2, B, 16), lambda i: (0, i, 0)),
          pl.BlockSpec((1, H), lambda i: (0, 0)),
          pl.BlockSpec((H, H), lambda i: (0, 0)),
      ],
      out_specs=pl.BlockSpec((B, H), lambda i: (i, 0)),
      out_shape=jax.ShapeDtypeStruct((N, H), jnp.float32),
  )(accp, y0, degp, b0, W1)


def _tc_out(accp, y1, degp, b1, batch3d, Wout, bout, G):
  N, H = y1.shape
  OUT = Wout.shape[1]
  B = 2000
  nblk = N // B

  def body(a_ref, y_ref, d_ref, b_ref, bt_ref, wo_ref,
           bo_ref, o_ref, pooled):
    i = pl.program_id(0)
    dis = _dis(d_ref)
    h = (a_ref[0] + a_ref[1] + y_ref[...]) * dis + b_ref[...]
    h = jnp.maximum(h, 0.0)
    seg = bt_ref[...].reshape(1, B)
    oh = (lax.broadcasted_iota(jnp.int32, (G, B), 0)
          == jnp.broadcast_to(seg, (G, B))).astype(jnp.float32)

    @pl.when(i == 0)
    def _():
      pooled[...] = jnp.zeros_like(pooled)

    pooled[...] += jnp.dot(oh, h, preferred_element_type=jnp.float32)

    @pl.when(i == nblk - 1)
    def _():
      o_ref[...] = jnp.dot(
          pooled[...], wo_ref[...],
          preferred_element_type=jnp.float32) + bo_ref[...]

  return pl.pallas_call(
      body, grid=(nblk,),
      in_specs=[
          pl.BlockSpec((2, B, H), lambda i: (0, i, 0)),
          pl.BlockSpec((B, H), lambda i: (i, 0)),
          pl.BlockSpec((2, B, 16), lambda i: (0, i, 0)),
          pl.BlockSpec((1, H), lambda i: (0, 0)),
          pl.BlockSpec((1, 1, B), lambda i: (i, 0, 0)),
          pl.BlockSpec((H, OUT), lambda i: (0, 0)),
          pl.BlockSpec((1, OUT), lambda i: (0, 0)),
      ],
      out_specs=pl.BlockSpec((G, OUT), lambda i: (0, 0)),
      out_shape=jax.ShapeDtypeStruct((G, OUT), jnp.float32),
      scratch_shapes=[pltpu.VMEM((G, OUT), jnp.float32)],
  )(accp, y1, degp, b1, batch3d, Wout, bout)


def kernel(x, edge_index, batch, W0, b0, W1, b1, Wout, bout):
  N, D = x.shape
  E = edge_index.shape[1]
  H = W0.shape[1]
  G = 64
  B = 2000

  CH = 100
  NCHUNK = E // NW // CH
  edge3 = edge_index.reshape(2, NW * NCHUNK, CH)

  degp = _deg_call(N, E)(edge3)
  y0 = _tc_in(x, W0, degp)
  acc = _agg_call(N, D, E)(y0, edge3)
  y1 = _tc_mid(acc, y0, degp, b0.reshape(1, H), W1)
  acc = _agg_call(N, H, E)(y1, edge3)
  return _tc_out(acc, y1, degp, b1.reshape(1, H),
                 batch.reshape(N // B, 1, B), Wout,
                 bout.reshape(1, Wout.shape[1]), G)


# speedup vs baseline: 1.2138x; 1.0411x over previous
import functools

import jax
import jax.numpy as jnp
from jax import lax
from jax.experimental import pallas as pl
from jax.experimental.pallas import tpu as pltpu
from jax.experimental.pallas import tpu_sc as plsc

NC = 2
NS = 16
NW = NC * NS


def _sc_mesh():
  return plsc.VectorSubcoreMesh(
      core_axis_name="c", subcore_axis_name="s", num_cores=NC,
      num_subcores=NS)


_SC_PARAMS = pltpu.CompilerParams(use_tc_tiling_on_sc=False)


@functools.lru_cache(maxsize=None)
def _deg_call(N, E):
  EW = E // NW
  CH = 100
  NCHUNK = EW // CH
  RPT = N // NS
  RCH = 125
  NR = RPT // RCH

  @functools.partial(
      pl.kernel, mesh=_sc_mesh(), compiler_params=_SC_PARAMS,
      out_type=jax.ShapeDtypeStruct((NC, N, 16), jnp.float32),
      scratch_types=[
          pltpu.VMEM_SHARED((N, 16), jnp.float32),
          pltpu.VMEM((NCHUNK, CH), jnp.int32),
          pltpu.VMEM((CH, 16), jnp.float32),
          pltpu.VMEM((RCH, 16), jnp.float32),
          pltpu.SemaphoreType.DMA,
          pltpu.SemaphoreType.DMA,
      ])
  def k(edge_hbm, out_hbm, acc, idx_v, ones_v, stage_v, sem_a, sem_b):
    c = lax.axis_index("c")
    s = lax.axis_index("s")
    wid = c * NS + s
    one = jnp.ones((16,), jnp.float32)
    for r in range(CH):
      ones_v[r, pl.ds(0, 16)] = one
    zero = jnp.zeros((16,), jnp.float32)
    for r in range(RCH):
      stage_v[r, pl.ds(0, 16)] = zero
    pltpu.sync_copy(edge_hbm.at[1, pl.ds(wid * NCHUNK, NCHUNK)], idx_v)
    for r in range(NR):
      pltpu.sync_copy(stage_v, acc.at[pl.ds(s * RPT + r * RCH, RCH)])
    plsc.subcore_barrier()

    def wait_s(sem):
      pltpu.make_async_copy(ones_v, acc.at[idx_v.at[0]], sem).wait()

    pltpu.async_copy(ones_v, acc.at[idx_v.at[0]], sem_a, add=True)

    def body(j2, carry):
      c0 = j2 * 2
      pltpu.async_copy(ones_v, acc.at[idx_v.at[c0 + 1]], sem_b, add=True)
      wait_s(sem_a)

      @pl.when(c0 + 2 < NCHUNK)
      def _():
        pltpu.async_copy(ones_v, acc.at[idx_v.at[c0 + 2]], sem_a, add=True)

      wait_s(sem_b)
      return carry

    lax.fori_loop(0, NCHUNK // 2, body, 0)
    if NCHUNK % 2:
      wait_s(sem_a)
    plsc.subcore_barrier()
    for r in range(NR):
      rr = s * RPT + r * RCH
      pltpu.sync_copy(acc.at[pl.ds(rr, RCH)], stage_v)
      pltpu.sync_copy(stage_v, out_hbm.at[c, pl.ds(rr, RCH)])

  return k


@functools.lru_cache(maxsize=None)
def _agg_call(N, D, E):
  EW = E // NW
  CH = 100
  NCHUNK = EW // CH
  RPT = N // NS
  RCH = 25
  NR = RPT // RCH

  @functools.partial(
      pl.kernel, mesh=_sc_mesh(), compiler_params=_SC_PARAMS,
      out_type=jax.ShapeDtypeStruct((NC, N, D), jnp.float32),
      scratch_types=[
          pltpu.VMEM_SHARED((N, D), jnp.float32),
          pltpu.VMEM((NCHUNK, CH), jnp.int32),
          pltpu.VMEM((NCHUNK, CH), jnp.int32),
          pltpu.VMEM((CH, D), jnp.float32),
          pltpu.VMEM((CH, D), jnp.float32),
          pltpu.VMEM((RCH, D), jnp.float32),
          pltpu.SemaphoreType.DMA,
          pltpu.SemaphoreType.DMA,
      ])
  def k(y_hbm, edge_hbm, out_hbm,
        acc, si_v, di_v, rows_a, rows_b, stage_v, sem_a, sem_b):
    c = lax.axis_index("c")
    s = lax.axis_index("s")
    wid = c * NS + s
    pltpu.sync_copy(edge_hbm.at[0, pl.ds(wid * NCHUNK, NCHUNK)], si_v)
    pltpu.sync_copy(edge_hbm.at[1, pl.ds(wid * NCHUNK, NCHUNK)], di_v)
    zero = jnp.zeros((16,), jnp.float32)
    for r in range(RCH):
      for l in range(D // 16):
        stage_v[r, pl.ds(l * 16, 16)] = zero
    for r in range(NR):
      pltpu.async_copy(stage_v, acc.at[pl.ds(s * RPT + r * RCH, RCH)], sem_a)
    for r in range(NR):
      pltpu.make_async_copy(
          stage_v, acc.at[pl.ds(s * RPT, RCH)], sem_a).wait()
    plsc.subcore_barrier()

    def wait_g(rows, sem):
      pltpu.make_async_copy(y_hbm.at[si_v.at[0]], rows, sem).wait()

    pltpu.async_copy(y_hbm.at[si_v.at[0]], rows_a, sem_a)

    def body(j2, carry):
      c0 = j2 * 2
      pltpu.async_copy(y_hbm.at[si_v.at[c0 + 1]], rows_b, sem_b)
      wait_g(rows_a, sem_a)
      pltpu.sync_copy(rows_a, acc.at[di_v.at[c0]], add=True)

      @pl.when(c0 + 2 < NCHUNK)
      def _():
        pltpu.async_copy(y_hbm.at[si_v.at[c0 + 2]], rows_a, sem_a)

      wait_g(rows_b, sem_b)
      pltpu.sync_copy(rows_b, acc.at[di_v.at[c0 + 1]], add=True)
      return carry

    lax.fori_loop(0, NCHUNK // 2, body, 0)
    if NCHUNK % 2:
      wait_g(rows_a, sem_a)
      pltpu.sync_copy(rows_a, acc.at[di_v.at[NCHUNK - 1]], add=True)
    plsc.subcore_barrier()

    sizes = [CH] * (RPT // CH) + ([RPT % CH] if RPT % CH else [])
    bufs = (rows_a, rows_b)
    sems = (sem_a, sem_b)
    offs = []
    off = s * RPT
    for sz in sizes:
      offs.append(off)
      off += sz
    for i, sz in enumerate(sizes):
      buf, sem = bufs[i % 2], sems[i % 2]
      if i >= 2:
        psz = sizes[i - 2]
        pltpu.make_async_copy(
            buf.at[pl.ds(0, psz)],
            out_hbm.at[c, pl.ds(offs[i - 2], psz)], sem).wait()
      pltpu.sync_copy(acc.at[pl.ds(offs[i], sz)], buf.at[pl.ds(0, sz)])
      pltpu.async_copy(
          buf.at[pl.ds(0, sz)], out_hbm.at[c, pl.ds(offs[i], sz)], sem)
    for i in range(max(0, len(sizes) - 2), len(sizes)):
      buf, sem, sz = bufs[i % 2], sems[i % 2], sizes[i]
      pltpu.make_async_copy(
          buf.at[pl.ds(0, sz)],
          out_hbm.at[c, pl.ds(offs[i], sz)], sem).wait()

  return k


def _dis(d_ref):
  return lax.rsqrt(d_ref[0, :, :1] + d_ref[1, :, :1] + 1.0)


def _tc_in(x, W0, degp):
  N, D = x.shape
  H = W0.shape[1]
  B = 2000

  def body(x_ref, w_ref, d_ref, o_ref):
    y = jnp.dot(x_ref[...], w_ref[...], preferred_element_type=jnp.float32)
    o_ref[...] = y * _dis(d_ref)

  return pl.pallas_call(
      body, grid=(N // B,),
      in_specs=[
          pl.BlockSpec((B, D), lambda i: (i, 0)),
          pl.BlockSpec((D, H), lambda i: (0, 0)),
          pl.BlockSpec((2, B, 16), lambda i: (0, i, 0)),
      ],
      out_specs=pl.BlockSpec((B, H), lambda i: (i, 0)),
      out_shape=jax.ShapeDtypeStruct((N, H), jnp.float32),
  )(x, W0, degp)


def _tc_mid(accp, y0, degp, b0, W1):
  N, H = y0.shape
  B = 2000

  def body(a_ref, y_ref, d_ref, b_ref, w_ref, o_ref):
    dis = _dis(d_ref)
    h = (a_ref[0] + a_ref[1] + y_ref[...]) * dis + b_ref[...]
    h = jnp.maximum(h, 0.0)
    o_ref[...] = jnp.dot(
        h, w_ref[...], preferred_element_type=jnp.float32) * dis

  return pl.pallas_call(
      body, grid=(N // B,),
      in_specs=[
          pl.BlockSpec((2, B, H), lambda i: (0, i, 0)),
          pl.BlockSpec((B, H), lambda i: (i, 0)),
          pl.BlockSpec((2, B, 16), lambda i: (0, i, 0)),
          pl.BlockSpec((1, H), lambda i: (0, 0)),
          pl.BlockSpec((H, H), lambda i: (0, 0)),
      ],
      out_specs=pl.BlockSpec((B, H), lambda i: (i, 0)),
      out_shape=jax.ShapeDtypeStruct((N, H), jnp.float32),
  )(accp, y0, degp, b0, W1)


def _tc_out(accp, y1, degp, b1, batch3d, Wout, bout, G):
  N, H = y1.shape
  OUT = Wout.shape[1]
  B = 2000
  nblk = N // B

  def body(a_ref, y_ref, d_ref, b_ref, bt_ref, wo_ref,
           bo_ref, o_ref, pooled):
    i = pl.program_id(0)
    dis = _dis(d_ref)
    h = (a_ref[0] + a_ref[1] + y_ref[...]) * dis + b_ref[...]
    h = jnp.maximum(h, 0.0)
    seg = bt_ref[...].reshape(1, B)
    oh = (lax.broadcasted_iota(jnp.int32, (G, B), 0)
          == jnp.broadcast_to(seg, (G, B))).astype(jnp.float32)

    @pl.when(i == 0)
    def _():
      pooled[...] = jnp.zeros_like(pooled)

    pooled[...] += jnp.dot(oh, h, preferred_element_type=jnp.float32)

    @pl.when(i == nblk - 1)
    def _():
      o_ref[...] = jnp.dot(
          pooled[...], wo_ref[...],
          preferred_element_type=jnp.float32) + bo_ref[...]

  return pl.pallas_call(
      body, grid=(nblk,),
      in_specs=[
          pl.BlockSpec((2, B, H), lambda i: (0, i, 0)),
          pl.BlockSpec((B, H), lambda i: (i, 0)),
          pl.BlockSpec((2, B, 16), lambda i: (0, i, 0)),
          pl.BlockSpec((1, H), lambda i: (0, 0)),
          pl.BlockSpec((1, 1, B), lambda i: (i, 0, 0)),
          pl.BlockSpec((H, OUT), lambda i: (0, 0)),
          pl.BlockSpec((1, OUT), lambda i: (0, 0)),
      ],
      out_specs=pl.BlockSpec((G, OUT), lambda i: (0, 0)),
      out_shape=jax.ShapeDtypeStruct((G, OUT), jnp.float32),
      scratch_shapes=[pltpu.VMEM((G, OUT), jnp.float32)],
  )(accp, y1, degp, b1, batch3d, Wout, bout)


def kernel(x, edge_index, batch, W0, b0, W1, b1, Wout, bout):
  N, D = x.shape
  E = edge_index.shape[1]
  H = W0.shape[1]
  G = 64
  B = 2000

  CH = 100
  NCHUNK = E // NW // CH
  edge3 = edge_index.reshape(2, NW * NCHUNK, CH)

  degp = _deg_call(N, E)(edge3)
  y0 = _tc_in(x, W0, degp)
  acc = _agg_call(N, D, E)(y0, edge3)
  y1 = _tc_mid(acc, y0, degp, b0.reshape(1, H), W1)
  acc = _agg_call(N, H, E)(y1, edge3)
  return _tc_out(acc, y1, degp, b1.reshape(1, H),
                 batch.reshape(N // B, 1, B), Wout,
                 bout.reshape(1, Wout.shape[1]), G)


# speedup vs baseline: 1.2180x; 1.0035x over previous
import functools

import jax
import jax.numpy as jnp
from jax import lax
from jax.experimental import pallas as pl
from jax.experimental.pallas import tpu as pltpu
from jax.experimental.pallas import tpu_sc as plsc

NC = 2
NS = 16
NW = NC * NS


def _sc_mesh():
  return plsc.VectorSubcoreMesh(
      core_axis_name="c", subcore_axis_name="s", num_cores=NC,
      num_subcores=NS)


_SC_PARAMS = pltpu.CompilerParams(use_tc_tiling_on_sc=False)


@functools.lru_cache(maxsize=None)
def _deg_call(N, E):
  EW = E // NW
  CH = 100
  NCHUNK = EW // CH
  RPT = N // NS
  RCH = 125
  NR = RPT // RCH

  @functools.partial(
      pl.kernel, mesh=_sc_mesh(), compiler_params=_SC_PARAMS,
      out_type=jax.ShapeDtypeStruct((NC, N, 16), jnp.float32),
      scratch_types=[
          pltpu.VMEM_SHARED((N, 16), jnp.float32),
          pltpu.VMEM((NCHUNK, CH), jnp.int32),
          pltpu.VMEM((CH, 16), jnp.float32),
          pltpu.VMEM((RCH, 16), jnp.float32),
          pltpu.VMEM((RCH, 16), jnp.float32),
          pltpu.SemaphoreType.DMA,
          pltpu.SemaphoreType.DMA,
      ])
  def k(edge_hbm, out_hbm, acc, idx_v, ones_v, stage_v, stage2_v,
        sem_a, sem_b):
    c = lax.axis_index("c")
    s = lax.axis_index("s")
    wid = c * NS + s
    one = jnp.ones((16,), jnp.float32)
    for r in range(CH):
      ones_v[r, pl.ds(0, 16)] = one
    zero = jnp.zeros((16,), jnp.float32)
    for r in range(RCH):
      stage_v[r, pl.ds(0, 16)] = zero
    pltpu.sync_copy(edge_hbm.at[1, pl.ds(wid * NCHUNK, NCHUNK)], idx_v)
    for r in range(NR):
      pltpu.async_copy(stage_v, acc.at[pl.ds(s * RPT + r * RCH, RCH)], sem_a)
    for r in range(NR):
      pltpu.make_async_copy(
          stage_v, acc.at[pl.ds(s * RPT, RCH)], sem_a).wait()
    plsc.subcore_barrier()

    def wait_s(sem):
      pltpu.make_async_copy(ones_v, acc.at[idx_v.at[0]], sem).wait()

    pltpu.async_copy(ones_v, acc.at[idx_v.at[0]], sem_a, add=True)

    def body(j2, carry):
      c0 = j2 * 2
      pltpu.async_copy(ones_v, acc.at[idx_v.at[c0 + 1]], sem_b, add=True)
      wait_s(sem_a)

      @pl.when(c0 + 2 < NCHUNK)
      def _():
        pltpu.async_copy(ones_v, acc.at[idx_v.at[c0 + 2]], sem_a, add=True)

      wait_s(sem_b)
      return carry

    lax.fori_loop(0, NCHUNK // 2, body, 0)
    if NCHUNK % 2:
      wait_s(sem_a)
    plsc.subcore_barrier()
    bufs = (stage_v, stage2_v)
    sems = (sem_a, sem_b)
    for r in range(NR):
      rr = s * RPT + r * RCH
      buf, sem = bufs[r % 2], sems[r % 2]
      if r >= 2:
        pltpu.make_async_copy(
            buf, out_hbm.at[c, pl.ds(s * RPT + (r - 2) * RCH, RCH)],
            sem).wait()
      pltpu.sync_copy(acc.at[pl.ds(rr, RCH)], buf)
      pltpu.async_copy(buf, out_hbm.at[c, pl.ds(rr, RCH)], sem)
    for r in range(max(0, NR - 2), NR):
      buf, sem = bufs[r % 2], sems[r % 2]
      pltpu.make_async_copy(
          buf, out_hbm.at[c, pl.ds(s * RPT + r * RCH, RCH)], sem).wait()

  return k


@functools.lru_cache(maxsize=None)
def _agg_call(N, D, E):
  EW = E // NW
  CH = 100
  NCHUNK = EW // CH
  RPT = N // NS
  RCH = 25
  NR = RPT // RCH

  @functools.partial(
      pl.kernel, mesh=_sc_mesh(), compiler_params=_SC_PARAMS,
      out_type=jax.ShapeDtypeStruct((NC, N, D), jnp.float32),
      scratch_types=[
          pltpu.VMEM_SHARED((N, D), jnp.float32),
          pltpu.VMEM((NCHUNK, CH), jnp.int32),
          pltpu.VMEM((NCHUNK, CH), jnp.int32),
          pltpu.VMEM((CH, D), jnp.float32),
          pltpu.VMEM((CH, D), jnp.float32),
          pltpu.VMEM((RCH, D), jnp.float32),
          pltpu.SemaphoreType.DMA,
          pltpu.SemaphoreType.DMA,
      ])
  def k(y_hbm, edge_hbm, out_hbm,
        acc, si_v, di_v, rows_a, rows_b, stage_v, sem_a, sem_b):
    c = lax.axis_index("c")
    s = lax.axis_index("s")
    wid = c * NS + s
    pltpu.sync_copy(edge_hbm.at[0, pl.ds(wid * NCHUNK, NCHUNK)], si_v)
    pltpu.sync_copy(edge_hbm.at[1, pl.ds(wid * NCHUNK, NCHUNK)], di_v)
    zero = jnp.zeros((16,), jnp.float32)
    for r in range(RCH):
      for l in range(D // 16):
        stage_v[r, pl.ds(l * 16, 16)] = zero
    for r in range(NR):
      pltpu.async_copy(stage_v, acc.at[pl.ds(s * RPT + r * RCH, RCH)], sem_a)
    for r in range(NR):
      pltpu.make_async_copy(
          stage_v, acc.at[pl.ds(s * RPT, RCH)], sem_a).wait()
    plsc.subcore_barrier()

    def wait_g(rows, sem):
      pltpu.make_async_copy(y_hbm.at[si_v.at[0]], rows, sem).wait()

    pltpu.async_copy(y_hbm.at[si_v.at[0]], rows_a, sem_a)

    def body(j2, carry):
      c0 = j2 * 2
      pltpu.async_copy(y_hbm.at[si_v.at[c0 + 1]], rows_b, sem_b)
      wait_g(rows_a, sem_a)
      pltpu.sync_copy(rows_a, acc.at[di_v.at[c0]], add=True)

      @pl.when(c0 + 2 < NCHUNK)
      def _():
        pltpu.async_copy(y_hbm.at[si_v.at[c0 + 2]], rows_a, sem_a)

      wait_g(rows_b, sem_b)
      pltpu.sync_copy(rows_b, acc.at[di_v.at[c0 + 1]], add=True)
      return carry

    lax.fori_loop(0, NCHUNK // 2, body, 0)
    if NCHUNK % 2:
      wait_g(rows_a, sem_a)
      pltpu.sync_copy(rows_a, acc.at[di_v.at[NCHUNK - 1]], add=True)
    plsc.subcore_barrier()

    sizes = [CH] * (RPT // CH) + ([RPT % CH] if RPT % CH else [])
    bufs = (rows_a, rows_b)
    sems = (sem_a, sem_b)
    offs = []
    off = s * RPT
    for sz in sizes:
      offs.append(off)
      off += sz
    for i, sz in enumerate(sizes):
      buf, sem = bufs[i % 2], sems[i % 2]
      if i >= 2:
        psz = sizes[i - 2]
        pltpu.make_async_copy(
            buf.at[pl.ds(0, psz)],
            out_hbm.at[c, pl.ds(offs[i - 2], psz)], sem).wait()
      pltpu.sync_copy(acc.at[pl.ds(offs[i], sz)], buf.at[pl.ds(0, sz)])
      pltpu.async_copy(
          buf.at[pl.ds(0, sz)], out_hbm.at[c, pl.ds(offs[i], sz)], sem)
    for i in range(max(0, len(sizes) - 2), len(sizes)):
      buf, sem, sz = bufs[i % 2], sems[i % 2], sizes[i]
      pltpu.make_async_copy(
          buf.at[pl.ds(0, sz)],
          out_hbm.at[c, pl.ds(offs[i], sz)], sem).wait()

  return k


def _dis(d_ref):
  return lax.rsqrt(d_ref[0, :, :1] + d_ref[1, :, :1] + 1.0)


def _tc_in(x, W0, degp):
  N, D = x.shape
  H = W0.shape[1]
  B = 2000

  def body(x_ref, w_ref, d_ref, o_ref):
    y = jnp.dot(x_ref[...], w_ref[...], preferred_element_type=jnp.float32)
    o_ref[...] = y * _dis(d_ref)

  return pl.pallas_call(
      body, grid=(N // B,),
      in_specs=[
          pl.BlockSpec((B, D), lambda i: (i, 0)),
          pl.BlockSpec((D, H), lambda i: (0, 0)),
          pl.BlockSpec((2, B, 16), lambda i: (0, i, 0)),
      ],
      out_specs=pl.BlockSpec((B, H), lambda i: (i, 0)),
      out_shape=jax.ShapeDtypeStruct((N, H), jnp.float32),
  )(x, W0, degp)


def _tc_mid(accp, y0, degp, b0, W1):
  N, H = y0.shape
  B = 2000

  def body(a_ref, y_ref, d_ref, b_ref, w_ref, o_ref):
    dis = _dis(d_ref)
    h = (a_ref[0] + a_ref[1] + y_ref[...]) * dis + b_ref[...]
    h = jnp.maximum(h, 0.0)
    o_ref[...] = jnp.dot(
        h, w_ref[...], preferred_element_type=jnp.float32) * dis

  return pl.pallas_call(
      body, grid=(N // B,),
      in_specs=[
          pl.BlockSpec((2, B, H), lambda i: (0, i, 0)),
          pl.BlockSpec((B, H), lambda i: (i, 0)),
          pl.BlockSpec((2, B, 16), lambda i: (0, i, 0)),
          pl.BlockSpec((1, H), lambda i: (0, 0)),
          pl.BlockSpec((H, H), lambda i: (0, 0)),
      ],
      out_specs=pl.BlockSpec((B, H), lambda i: (i, 0)),
      out_shape=jax.ShapeDtypeStruct((N, H), jnp.float32),
  )(accp, y0, degp, b0, W1)


def _tc_out(accp, y1, degp, b1, batch3d, Wout, bout, G):
  N, H = y1.shape
  OUT = Wout.shape[1]
  B = 2000
  nblk = N // B

  def body(a_ref, y_ref, d_ref, b_ref, bt_ref, wo_ref,
           bo_ref, o_ref, pooled):
    i = pl.program_id(0)
    dis = _dis(d_ref)
    h = (a_ref[0] + a_ref[1] + y_ref[...]) * dis + b_ref[...]
    h = jnp.maximum(h, 0.0)
    seg = bt_ref[...].reshape(1, B)
    oh = (lax.broadcasted_iota(jnp.int32, (G, B), 0)
          == jnp.broadcast_to(seg, (G, B))).astype(jnp.float32)

    @pl.when(i == 0)
    def _():
      pooled[...] = jnp.zeros_like(pooled)

    pooled[...] += jnp.dot(oh, h, preferred_element_type=jnp.float32)

    @pl.when(i == nblk - 1)
    def _():
      o_ref[...] = jnp.dot(
          pooled[...], wo_ref[...],
          preferred_element_type=jnp.float32) + bo_ref[...]

  return pl.pallas_call(
      body, grid=(nblk,),
      in_specs=[
          pl.BlockSpec((2, B, H), lambda i: (0, i, 0)),
          pl.BlockSpec((B, H), lambda i: (i, 0)),
          pl.BlockSpec((2, B, 16), lambda i: (0, i, 0)),
          pl.BlockSpec((1, H), lambda i: (0, 0)),
          pl.BlockSpec((1, 1, B), lambda i: (i, 0, 0)),
          pl.BlockSpec((H, OUT), lambda i: (0, 0)),
          pl.BlockSpec((1, OUT), lambda i: (0, 0)),
      ],
      out_specs=pl.BlockSpec((G, OUT), lambda i: (0, 0)),
      out_shape=jax.ShapeDtypeStruct((G, OUT), jnp.float32),
      scratch_shapes=[pltpu.VMEM((G, OUT), jnp.float32)],
  )(accp, y1, degp, b1, batch3d, Wout, bout)


def kernel(x, edge_index, batch, W0, b0, W1, b1, Wout, bout):
  N, D = x.shape
  E = edge_index.shape[1]
  H = W0.shape[1]
  G = 64
  B = 2000

  CH = 100
  NCHUNK = E // NW // CH
  edge3 = edge_index.reshape(2, NW * NCHUNK, CH)

  degp = _deg_call(N, E)(edge3)
  y0 = _tc_in(x, W0, degp)
  acc = _agg_call(N, D, E)(y0, edge3)
  y1 = _tc_mid(acc, y0, degp, b0.reshape(1, H), W1)
  acc = _agg_call(N, H, E)(y1, edge3)
  return _tc_out(acc, y1, degp, b1.reshape(1, H),
                 batch.reshape(N // B, 1, B), Wout,
                 bout.reshape(1, Wout.shape[1]), G)
